# baseline (speedup 1.0000x reference)
import functools

import jax
import jax.numpy as jnp
from jax import lax
from jax.experimental import pallas as pl
from jax.experimental.pallas import tpu as pltpu
from jax.experimental.pallas import tpu_sc as plsc

N = 10000
E = 320000
D = 128

NC = 2
NS = 16
NW = NC * NS
EDGES_PER_TILE = E // NW
CHUNK = 128
NCHUNK = 80
E_PAD = NW * NCHUNK * CHUNK
NPAD = 10240
ROWS_PER_TILE = NPAD // NS

_sc_scatter_cache = []


def _get_sc_scatter():
    if _sc_scatter_cache:
        return _sc_scatter_cache[0]

    mesh = plsc.VectorSubcoreMesh(core_axis_name="c", subcore_axis_name="s")

    @functools.partial(
        pl.kernel,
        mesh=mesh,
        out_type=jax.ShapeDtypeStruct((NC, NPAD, D), jnp.float32),
        scratch_types=[
            pltpu.VMEM((NCHUNK, CHUNK), jnp.int32),
            pltpu.VMEM((NCHUNK, CHUNK), jnp.int32),
            pltpu.VMEM((CHUNK, D), jnp.float32),
            pltpu.VMEM_SHARED((NPAD, D), jnp.float32),
            pltpu.SemaphoreType.DMA,
        ],
    )
    def _sc_scatter(row_hbm, col_hbm, feat_hbm, zeros_hbm, partial_hbm,
                    row_v, col_v, msg_v, agg_sh, sem):
        c = lax.axis_index("c")
        s = lax.axis_index("s")
        wid = s * NC + c

        pltpu.sync_copy(row_hbm.at[wid], row_v)
        pltpu.sync_copy(col_hbm.at[wid], col_v)

        base = s * ROWS_PER_TILE
        pltpu.sync_copy(zeros_hbm.at[pl.ds(base, ROWS_PER_TILE)],
                        agg_sh.at[pl.ds(base, ROWS_PER_TILE)])
        plsc.subcore_barrier()

        def body(j, carry):
            pltpu.async_copy(feat_hbm.at[col_v.at[j]], msg_v, sem).wait()
            pltpu.sync_copy(msg_v, agg_sh.at[row_v.at[j]], add=True)
            return carry

        lax.fori_loop(0, NCHUNK, body, 0)
        plsc.subcore_barrier()

        pltpu.sync_copy(agg_sh.at[pl.ds(base, ROWS_PER_TILE)],
                        partial_hbm.at[c].at[pl.ds(base, ROWS_PER_TILE)])

    _sc_scatter_cache.append(_sc_scatter)
    return _sc_scatter


def _mm_body(x_ref, w_ref, b_ref, o_ref):
    o_ref[...] = (
        jnp.dot(x_ref[...], w_ref[...], preferred_element_type=jnp.float32)
        + b_ref[...]
    )


def _linear(x, W, b):
    m_blk = 1000
    grid = (N // m_blk,)
    return pl.pallas_call(
        _mm_body,
        grid=grid,
        in_specs=[
            pl.BlockSpec((m_blk, D), lambda i: (i, 0)),
            pl.BlockSpec((D, D), lambda i: (0, 0)),
            pl.BlockSpec((1, D), lambda i: (0, 0)),
        ],
        out_specs=pl.BlockSpec((m_blk, D), lambda i: (i, 0)),
        out_shape=jax.ShapeDtypeStruct((N, D), jnp.float32),
    )(x, W, b.reshape(1, D))


def _add_body(o_ref, p0_ref, p1_ref, f_ref):
    f_ref[...] = o_ref[...] + p0_ref[...] + p1_ref[...]


def _final_add(out, p0, p1):
    m_blk = 1000
    grid = (N // m_blk,)
    spec = pl.BlockSpec((m_blk, D), lambda i: (i, 0))
    return pl.pallas_call(
        _add_body,
        grid=grid,
        in_specs=[spec, spec, spec],
        out_specs=spec,
        out_shape=jax.ShapeDtypeStruct((N, D), jnp.float32),
    )(out, p0, p1)


def kernel(x, edge_index, W, b):
    out = _linear(x, W, b)
    pad = E_PAD - E
    row = jnp.concatenate(
        [edge_index[0], jnp.full((pad,), N, jnp.int32)]).reshape(
            NW, NCHUNK, CHUNK)
    col = jnp.concatenate(
        [edge_index[1], jnp.zeros((pad,), jnp.int32)]).reshape(
            NW, NCHUNK, CHUNK)
    zeros = jnp.zeros((NPAD, D), jnp.float32)
    partial = _get_sc_scatter()(row, col, out, zeros)
    return _final_add(out, partial[0, :N], partial[1, :N])


# speedup vs baseline: 1.0110x; 1.0110x over previous
import functools

import jax
import jax.numpy as jnp
from jax import lax
from jax.experimental import pallas as pl
from jax.experimental.pallas import tpu as pltpu
from jax.experimental.pallas import tpu_sc as plsc

N = 10000
E = 320000
D = 128

NC = 2
NS = 16
NW = NC * NS
EDGES_PER_TILE = E // NW
CHUNK = 64
NCHUNK = 160
E_PAD = NW * NCHUNK * CHUNK
NPAD = 10240
ROWS_PER_TILE = NPAD // NS

_sc_scatter_cache = []


def _get_sc_scatter():
    if _sc_scatter_cache:
        return _sc_scatter_cache[0]

    mesh = plsc.VectorSubcoreMesh(core_axis_name="c", subcore_axis_name="s")

    @functools.partial(
        pl.kernel,
        mesh=mesh,
        out_type=jax.ShapeDtypeStruct((NC, NPAD, D), jnp.float32),
        scratch_types=[
            pltpu.VMEM((NCHUNK, CHUNK), jnp.int32),
            pltpu.VMEM((NCHUNK, CHUNK), jnp.int32),
            pltpu.VMEM((CHUNK, D), jnp.float32),
            pltpu.VMEM_SHARED((NPAD, D), jnp.float32),
            pltpu.SemaphoreType.DMA,
        ],
    )
    def _sc_scatter(row_hbm, col_hbm, feat_hbm, zeros_hbm, partial_hbm,
                    row_v, col_v, msg_v, agg_sh, sem):
        c = lax.axis_index("c")
        s = lax.axis_index("s")
        wid = s * NC + c

        pltpu.sync_copy(row_hbm.at[wid], row_v)
        pltpu.sync_copy(col_hbm.at[wid], col_v)

        base = s * ROWS_PER_TILE
        pltpu.sync_copy(zeros_hbm.at[pl.ds(base, ROWS_PER_TILE)],
                        agg_sh.at[pl.ds(base, ROWS_PER_TILE)])
        plsc.subcore_barrier()

        def body(j, carry):
            pltpu.async_copy(feat_hbm.at[col_v.at[j]], msg_v, sem).wait()
            pltpu.sync_copy(msg_v, agg_sh.at[row_v.at[j]], add=True)
            return carry

        lax.fori_loop(0, NCHUNK, body, 0)
        plsc.subcore_barrier()

        pltpu.sync_copy(agg_sh.at[pl.ds(base, ROWS_PER_TILE)],
                        partial_hbm.at[c].at[pl.ds(base, ROWS_PER_TILE)])

    _sc_scatter_cache.append(_sc_scatter)
    return _sc_scatter


def _mm_body(x_ref, w_ref, b_ref, o_ref):
    o_ref[...] = (
        jnp.dot(x_ref[...], w_ref[...], preferred_element_type=jnp.float32)
        + b_ref[...]
    )


def _linear(x, W, b):
    m_blk = 1000
    grid = (N // m_blk,)
    return pl.pallas_call(
        _mm_body,
        grid=grid,
        in_specs=[
            pl.BlockSpec((m_blk, D), lambda i: (i, 0)),
            pl.BlockSpec((D, D), lambda i: (0, 0)),
            pl.BlockSpec((1, D), lambda i: (0, 0)),
        ],
        out_specs=pl.BlockSpec((m_blk, D), lambda i: (i, 0)),
        out_shape=jax.ShapeDtypeStruct((N, D), jnp.float32),
    )(x, W, b.reshape(1, D))


def _add_body(o_ref, p0_ref, p1_ref, f_ref):
    f_ref[...] = o_ref[...] + p0_ref[...] + p1_ref[...]


def _final_add(out, p0, p1):
    m_blk = 1000
    grid = (N // m_blk,)
    spec = pl.BlockSpec((m_blk, D), lambda i: (i, 0))
    return pl.pallas_call(
        _add_body,
        grid=grid,
        in_specs=[spec, spec, spec],
        out_specs=spec,
        out_shape=jax.ShapeDtypeStruct((N, D), jnp.float32),
    )(out, p0, p1)


def kernel(x, edge_index, W, b):
    out = _linear(x, W, b)
    pad = E_PAD - E
    row = jnp.concatenate(
        [edge_index[0], jnp.full((pad,), N, jnp.int32)]).reshape(
            NW, NCHUNK, CHUNK)
    col = jnp.concatenate(
        [edge_index[1], jnp.zeros((pad,), jnp.int32)]).reshape(
            NW, NCHUNK, CHUNK)
    zeros = jnp.zeros((NPAD, D), jnp.float32)
    partial = _get_sc_scatter()(row, col, out, zeros)
    return _final_add(out, partial[0, :N], partial[1, :N])


# speedup vs baseline: 2.4136x; 2.3872x over previous
import functools

import jax
import jax.numpy as jnp
from jax import lax
from jax.experimental import pallas as pl
from jax.experimental.pallas import tpu as pltpu
from jax.experimental.pallas import tpu_sc as plsc

N = 10000
E = 320000
D = 128

NC = 2
NS = 16
NW = NC * NS
EDGES_PER_TILE = E // NW
CHUNK = 128
NCHUNK = 80
E_PAD = NW * NCHUNK * CHUNK
NPAD = 10240
ROWS_PER_TILE = NPAD // NS

_sc_scatter_cache = []


def _get_sc_scatter():
    if _sc_scatter_cache:
        return _sc_scatter_cache[0]

    mesh = plsc.VectorSubcoreMesh(core_axis_name="c", subcore_axis_name="s")

    @functools.partial(
        pl.kernel,
        mesh=mesh,
        out_type=jax.ShapeDtypeStruct((NC, NPAD, D), jnp.float32),
        scratch_types=[
            pltpu.VMEM((NCHUNK, CHUNK), jnp.int32),
            pltpu.VMEM((NCHUNK, CHUNK), jnp.int32),
            pltpu.VMEM((CHUNK, D), jnp.float32),
            pltpu.VMEM_SHARED((NPAD, D), jnp.float32),
            pltpu.SemaphoreType.DMA,
        ],
    )
    def _sc_scatter(row_hbm, col_hbm, feat_hbm, zeros_hbm, partial_hbm,
                    row_v, col_v, msg_v, agg_sh, sem):
        c = lax.axis_index("c")
        s = lax.axis_index("s")
        wid = s * NC + c

        pltpu.sync_copy(row_hbm.at[wid], row_v)
        pltpu.sync_copy(col_hbm.at[wid], col_v)

        base = s * ROWS_PER_TILE
        pltpu.sync_copy(zeros_hbm.at[pl.ds(base, ROWS_PER_TILE)],
                        agg_sh.at[pl.ds(base, ROWS_PER_TILE)])
        plsc.subcore_barrier()

        def body(j, carry):
            pltpu.async_copy(feat_hbm.at[col_v.at[j]], msg_v, sem).wait()
            pltpu.sync_copy(msg_v, agg_sh.at[row_v.at[j]], add=True)
            return carry

        lax.fori_loop(0, NCHUNK, body, 0)
        plsc.subcore_barrier()

        pltpu.sync_copy(agg_sh.at[pl.ds(base, ROWS_PER_TILE)],
                        partial_hbm.at[c].at[pl.ds(base, ROWS_PER_TILE)])

    _sc_scatter_cache.append(_sc_scatter)
    return _sc_scatter


def _mm_body(x_ref, w_ref, b_ref, o_ref):
    o_ref[...] = (
        jnp.dot(x_ref[...], w_ref[...], preferred_element_type=jnp.float32)
        + b_ref[...]
    )


def _linear(x, W, b):
    m_blk = 1000
    grid = (N // m_blk,)
    return pl.pallas_call(
        _mm_body,
        grid=grid,
        in_specs=[
            pl.BlockSpec((m_blk, D), lambda i: (i, 0)),
            pl.BlockSpec((D, D), lambda i: (0, 0)),
            pl.BlockSpec((1, D), lambda i: (0, 0)),
        ],
        out_specs=pl.BlockSpec((m_blk, D), lambda i: (i, 0)),
        out_shape=jax.ShapeDtypeStruct((N, D), jnp.float32),
    )(x, W, b.reshape(1, D))


def _add_body(o_ref, p0_ref, p1_ref, f_ref):
    f_ref[...] = o_ref[...] + p0_ref[...] + p1_ref[...]


def _final_add(out, p0, p1):
    m_blk = 1000
    grid = (N // m_blk,)
    spec = pl.BlockSpec((m_blk, D), lambda i: (i, 0))
    return pl.pallas_call(
        _add_body,
        grid=grid,
        in_specs=[spec, spec, spec],
        out_specs=spec,
        out_shape=jax.ShapeDtypeStruct((N, D), jnp.float32),
    )(out, p0, p1)


def kernel(x, edge_index, W, b):
    out = _linear(x, W, b)
    pad_per_tile = NCHUNK * CHUNK - E // NW
    pad_rows = jnp.broadcast_to(N + jnp.arange(pad_per_tile, dtype=jnp.int32),
                                (NW, pad_per_tile))
    pad_cols = jnp.broadcast_to(jnp.arange(pad_per_tile, dtype=jnp.int32),
                                (NW, pad_per_tile))
    row = jnp.concatenate(
        [edge_index[0].reshape(NW, E // NW), pad_rows], axis=1).reshape(
            NW, NCHUNK, CHUNK)
    col = jnp.concatenate(
        [edge_index[1].reshape(NW, E // NW), pad_cols], axis=1).reshape(
            NW, NCHUNK, CHUNK)
    zeros = jnp.zeros((NPAD, D), jnp.float32)
    partial = _get_sc_scatter()(row, col, out, zeros)
    return _final_add(out, partial[0, :N], partial[1, :N])


# speedup vs baseline: 3.0231x; 1.2525x over previous
import functools

import jax
import jax.numpy as jnp
from jax import lax
from jax.experimental import pallas as pl
from jax.experimental.pallas import tpu as pltpu
from jax.experimental.pallas import tpu_sc as plsc

N = 10000
E = 320000
D = 128

NC = 2
NS = 16
NW = NC * NS
EDGES_PER_TILE = E // NW
CHUNK = 128
NCHUNK = 80
IB = 8
NB = NCHUNK // IB
E_PAD = NW * NCHUNK * CHUNK
NPAD = 10240
ROWS_PER_TILE = NPAD // NS

_sc_scatter_cache = []


def _get_sc_scatter():
    if _sc_scatter_cache:
        return _sc_scatter_cache[0]

    mesh = plsc.VectorSubcoreMesh(core_axis_name="c", subcore_axis_name="s")

    @functools.partial(
        pl.kernel,
        mesh=mesh,
        out_type=jax.ShapeDtypeStruct((NC, NPAD, D), jnp.float32),
        scratch_types=[
            pltpu.VMEM((2, IB, CHUNK), jnp.int32),
            pltpu.VMEM((2, IB, CHUNK), jnp.int32),
            pltpu.VMEM((CHUNK, D), jnp.float32),
            pltpu.VMEM((CHUNK, D), jnp.float32),
            pltpu.VMEM_SHARED((NPAD, D), jnp.float32),
            pltpu.SemaphoreType.DMA,
            pltpu.SemaphoreType.DMA,
            pltpu.SemaphoreType.DMA,
            pltpu.SemaphoreType.DMA,
            pltpu.SemaphoreType.DMA,
        ],
    )
    def _sc_scatter(row_hbm, col_hbm, feat_hbm, zeros_hbm, partial_hbm,
                    rowb, colb, msg_a, msg_b, agg_sh,
                    isem, gsem_a, gsem_b, ssem_a, ssem_b):
        c = lax.axis_index("c")
        s = lax.axis_index("s")
        wid = s * NC + c
        row_t = row_hbm.at[wid]
        col_t = col_hbm.at[wid]

        msgs = (msg_a, msg_b)
        gsems = (gsem_a, gsem_b)
        ssems = (ssem_a, ssem_b)

        def idx_start(kb, slot):
            pltpu.async_copy(row_t.at[pl.ds(kb * IB, IB)], rowb.at[slot], isem)
            pltpu.async_copy(col_t.at[pl.ds(kb * IB, IB)], colb.at[slot], isem)

        def idx_wait(kb, slot):
            pltpu.make_async_copy(row_t.at[pl.ds(kb * IB, IB)],
                                  rowb.at[slot], isem).wait()
            pltpu.make_async_copy(col_t.at[pl.ds(kb * IB, IB)],
                                  colb.at[slot], isem).wait()

        def gstart(slot, i, bi):
            pltpu.async_copy(feat_hbm.at[colb.at[slot].at[i]], msgs[bi],
                             gsems[bi])

        def gwait(slot, i, bi):
            pltpu.make_async_copy(feat_hbm.at[colb.at[slot].at[i]], msgs[bi],
                                  gsems[bi]).wait()

        def sstart(slot, i, bi):
            pltpu.async_copy(msgs[bi], agg_sh.at[rowb.at[slot].at[i]],
                             ssems[bi], add=True)

        def swait(slot, i, bi):
            pltpu.make_async_copy(msgs[bi], agg_sh.at[rowb.at[slot].at[i]],
                                  ssems[bi]).wait()

        base = s * ROWS_PER_TILE
        pltpu.sync_copy(zeros_hbm.at[pl.ds(base, ROWS_PER_TILE)],
                        agg_sh.at[pl.ds(base, ROWS_PER_TILE)])

        idx_start(0, 0)
        idx_wait(0, 0)
        gstart(0, 0, 0)
        plsc.subcore_barrier()

        def outer(kb, carry):
            p = lax.rem(kb, 2)
            q = 1 - p
            for i in range(IB):
                bi = i % 2
                bo = 1 - bi
                gwait(p, i, bi)
                sstart(p, i, bi)
                if i == 0:
                    @pl.when(kb > 0)
                    def _():
                        swait(q, IB - 1, bo)
                elif i == 1:
                    swait(p, i - 1, bo)

                    @pl.when(kb + 1 < NB)
                    def _():
                        idx_start(kb + 1, q)
                else:
                    swait(p, i - 1, bo)
                if i + 1 < IB:
                    gstart(p, i + 1, bo)
                else:
                    @pl.when(kb + 1 < NB)
                    def _():
                        idx_wait(kb + 1, q)
                        gstart(q, 0, bo)
            return carry

        lax.fori_loop(0, NB, outer, 0)
        swait((NB - 1) % 2, IB - 1, (IB - 1) % 2)
        plsc.subcore_barrier()

        pltpu.sync_copy(agg_sh.at[pl.ds(base, ROWS_PER_TILE)],
                        partial_hbm.at[c].at[pl.ds(base, ROWS_PER_TILE)])

    _sc_scatter_cache.append(_sc_scatter)
    return _sc_scatter


def _mm_body(x_ref, w_ref, b_ref, o_ref):
    o_ref[...] = (
        jnp.dot(x_ref[...], w_ref[...], preferred_element_type=jnp.float32)
        + b_ref[...]
    )


def _linear(x, W, b):
    m_blk = 1000
    grid = (N // m_blk,)
    return pl.pallas_call(
        _mm_body,
        grid=grid,
        in_specs=[
            pl.BlockSpec((m_blk, D), lambda i: (i, 0)),
            pl.BlockSpec((D, D), lambda i: (0, 0)),
            pl.BlockSpec((1, D), lambda i: (0, 0)),
        ],
        out_specs=pl.BlockSpec((m_blk, D), lambda i: (i, 0)),
        out_shape=jax.ShapeDtypeStruct((N, D), jnp.float32),
    )(x, W, b.reshape(1, D))


def _add_body(o_ref, p0_ref, p1_ref, f_ref):
    f_ref[...] = o_ref[...] + p0_ref[...] + p1_ref[...]


def _final_add(out, p0, p1):
    m_blk = 1000
    grid = (N // m_blk,)
    spec = pl.BlockSpec((m_blk, D), lambda i: (i, 0))
    return pl.pallas_call(
        _add_body,
        grid=grid,
        in_specs=[spec, spec, spec],
        out_specs=spec,
        out_shape=jax.ShapeDtypeStruct((N, D), jnp.float32),
    )(out, p0, p1)


def kernel(x, edge_index, W, b):
    out = _linear(x, W, b)
    pad_per_tile = NCHUNK * CHUNK - E // NW
    pad_rows = jnp.broadcast_to(N + jnp.arange(pad_per_tile, dtype=jnp.int32),
                                (NW, pad_per_tile))
    pad_cols = jnp.broadcast_to(jnp.arange(pad_per_tile, dtype=jnp.int32),
                                (NW, pad_per_tile))
    row = jnp.concatenate(
        [edge_index[0].reshape(NW, E // NW), pad_rows], axis=1).reshape(
            NW, NCHUNK, CHUNK)
    col = jnp.concatenate(
        [edge_index[1].reshape(NW, E // NW), pad_cols], axis=1).reshape(
            NW, NCHUNK, CHUNK)
    zeros = jnp.zeros((NPAD, D), jnp.float32)
    partial = _get_sc_scatter()(row, col, out, zeros)
    return _final_add(out, partial[0, :N], partial[1, :N])


# speedup vs baseline: 3.1517x; 1.0425x over previous
import functools

import jax
import jax.numpy as jnp
from jax import lax
from jax.experimental import pallas as pl
from jax.experimental.pallas import tpu as pltpu
from jax.experimental.pallas import tpu_sc as plsc

N = 10000
E = 320000
D = 128

NC = 2
NS = 16
NW = NC * NS
EDGES_PER_TILE = E // NW
CHUNK = 125
NCHUNK = 80
IB = 8
NB = NCHUNK // IB
NPAD = 10240
ROWS_PER_TILE = NPAD // NS

_sc_scatter_cache = []


def _get_sc_scatter():
    if _sc_scatter_cache:
        return _sc_scatter_cache[0]

    mesh = plsc.VectorSubcoreMesh(core_axis_name="c", subcore_axis_name="s")

    @functools.partial(
        pl.kernel,
        mesh=mesh,
        out_type=jax.ShapeDtypeStruct((NC, NPAD, D), jnp.float32),
        scratch_types=[
            pltpu.VMEM((2, IB, CHUNK), jnp.int32),
            pltpu.VMEM((2, IB, CHUNK), jnp.int32),
            pltpu.VMEM((CHUNK, D), jnp.float32),
            pltpu.VMEM((CHUNK, D), jnp.float32),
            pltpu.VMEM_SHARED((NPAD, D), jnp.float32),
            pltpu.SemaphoreType.DMA,
            pltpu.SemaphoreType.DMA,
            pltpu.SemaphoreType.DMA,
            pltpu.SemaphoreType.DMA,
            pltpu.SemaphoreType.DMA,
        ],
    )
    def _sc_scatter(row_hbm, col_hbm, feat_hbm, zeros_hbm, partial_hbm,
                    rowb, colb, msg_a, msg_b, agg_sh,
                    isem, gsem_a, gsem_b, ssem_a, ssem_b):
        c = lax.axis_index("c")
        s = lax.axis_index("s")
        wid = s * NC + c
        row_t = row_hbm.at[wid]
        col_t = col_hbm.at[wid]

        msgs = (msg_a, msg_b)
        gsems = (gsem_a, gsem_b)
        ssems = (ssem_a, ssem_b)

        def idx_start(kb, slot):
            pltpu.async_copy(row_t.at[pl.ds(kb * IB, IB)], rowb.at[slot], isem)
            pltpu.async_copy(col_t.at[pl.ds(kb * IB, IB)], colb.at[slot], isem)

        def idx_wait(kb, slot):
            pltpu.make_async_copy(row_t.at[pl.ds(kb * IB, IB)],
                                  rowb.at[slot], isem).wait()
            pltpu.make_async_copy(col_t.at[pl.ds(kb * IB, IB)],
                                  colb.at[slot], isem).wait()

        def gstart(slot, i, bi):
            pltpu.async_copy(feat_hbm.at[colb.at[slot].at[i]], msgs[bi],
                             gsems[bi])

        def gwait(slot, i, bi):
            pltpu.make_async_copy(feat_hbm.at[colb.at[slot].at[i]], msgs[bi],
                                  gsems[bi]).wait()

        def sstart(slot, i, bi):
            pltpu.async_copy(msgs[bi], agg_sh.at[rowb.at[slot].at[i]],
                             ssems[bi], add=True)

        def swait(slot, i, bi):
            pltpu.make_async_copy(msgs[bi], agg_sh.at[rowb.at[slot].at[i]],
                                  ssems[bi]).wait()

        base = s * ROWS_PER_TILE
        pltpu.sync_copy(zeros_hbm, agg_sh.at[pl.ds(base, ROWS_PER_TILE)])

        idx_start(0, 0)
        idx_wait(0, 0)
        gstart(0, 0, 0)
        plsc.subcore_barrier()

        def outer(kb, carry):
            p = lax.rem(kb, 2)
            q = 1 - p
            for i in range(IB):
                bi = i % 2
                bo = 1 - bi
                gwait(p, i, bi)
                sstart(p, i, bi)
                if i == 0:
                    @pl.when(kb > 0)
                    def _():
                        swait(q, IB - 1, bo)
                elif i == 1:
                    swait(p, i - 1, bo)

                    @pl.when(kb + 1 < NB)
                    def _():
                        idx_start(kb + 1, q)
                else:
                    swait(p, i - 1, bo)
                if i + 1 < IB:
                    gstart(p, i + 1, bo)
                else:
                    @pl.when(kb + 1 < NB)
                    def _():
                        idx_wait(kb + 1, q)
                        gstart(q, 0, bo)
            return carry

        lax.fori_loop(0, NB, outer, 0)
        swait((NB - 1) % 2, IB - 1, (IB - 1) % 2)
        plsc.subcore_barrier()

        pltpu.sync_copy(agg_sh.at[pl.ds(base, ROWS_PER_TILE)],
                        partial_hbm.at[c].at[pl.ds(base, ROWS_PER_TILE)])

    _sc_scatter_cache.append(_sc_scatter)
    return _sc_scatter


def _mm_body(x_ref, w_ref, b_ref, o_ref):
    o_ref[...] = (
        jnp.dot(x_ref[...], w_ref[...], preferred_element_type=jnp.float32)
        + b_ref[...]
    )


def _linear(x, W, b):
    m_blk = 1000
    grid = (N // m_blk,)
    return pl.pallas_call(
        _mm_body,
        grid=grid,
        in_specs=[
            pl.BlockSpec((m_blk, D), lambda i: (i, 0)),
            pl.BlockSpec((D, D), lambda i: (0, 0)),
            pl.BlockSpec((1, D), lambda i: (0, 0)),
        ],
        out_specs=pl.BlockSpec((m_blk, D), lambda i: (i, 0)),
        out_shape=jax.ShapeDtypeStruct((N, D), jnp.float32),
    )(x, W, b.reshape(1, D))


def _add_body(o_ref, p0_ref, p1_ref, f_ref):
    f_ref[...] = o_ref[...] + p0_ref[0] + p1_ref[0]


def _final_add(out, partial):
    m_blk = 1000
    grid = (N // m_blk,)
    spec = pl.BlockSpec((m_blk, D), lambda i: (i, 0))
    return pl.pallas_call(
        _add_body,
        grid=grid,
        in_specs=[
            spec,
            pl.BlockSpec((1, m_blk, D), lambda i: (0, i, 0)),
            pl.BlockSpec((1, m_blk, D), lambda i: (1, i, 0)),
        ],
        out_specs=spec,
        out_shape=jax.ShapeDtypeStruct((N, D), jnp.float32),
    )(out, partial, partial)


def kernel(x, edge_index, W, b):
    out = _linear(x, W, b)
    row = edge_index[0].reshape(NW, NCHUNK, CHUNK)
    col = edge_index[1].reshape(NW, NCHUNK, CHUNK)
    zeros = jnp.zeros((ROWS_PER_TILE, D), jnp.float32)
    partial = _get_sc_scatter()(row, col, out, zeros)
    return _final_add(out, partial)
